# Initial kernel scaffold; baseline (speedup 1.0000x reference)
import jax
import jax.numpy as jnp
from jax.experimental import pallas as pl


def kernel(x, edge_index, W1, b1, W2, b2):
    raise NotImplementedError("write your pallas kernel here")


# speedup vs baseline: 17.5381x; 17.5381x over previous
import functools

import jax
import jax.numpy as jnp
from jax import lax
from jax.experimental import pallas as pl
from jax.experimental.pallas import tpu as pltpu
from jax.experimental.pallas import tpu_sc as plsc

N = 10000
F = 64
K = 10
ALPHA = 0.1

N_PAD = 10240
NPT = 640
NCH = 5
RC = 128

TILES = 16
EC = 128
ECH = 162
NBUF = 3
EGRP = ECH // NBUF
E_SLOTS = TILES * ECH * EC

_mesh = plsc.VectorSubcoreMesh(core_axis_name="c", subcore_axis_name="s")


def _rsqrt16(v):
    ii = plsc.bitcast(v, jnp.int32)
    ii = 0x5F3759DF - lax.shift_right_logical(ii, 1)
    y = plsc.bitcast(ii, jnp.float32)
    for _ in range(3):
        y = y * (1.5 - 0.5 * v * y * y)
    return y


@functools.partial(
    pl.kernel,
    out_type=(
        jax.ShapeDtypeStruct((N_PAD, F), jnp.float32),
        jax.ShapeDtypeStruct((N_PAD, F), jnp.float32),
        jax.ShapeDtypeStruct((N_PAD, F), jnp.float32),
    ),
    mesh=_mesh,
    compiler_params=pltpu.CompilerParams(needs_layout_passes=False,
                                         use_tc_tiling_on_sc=False),
    scratch_types=[
        pltpu.VMEM((2, NBUF, EC), jnp.int32),
        pltpu.VMEM((2, NBUF, EC), jnp.int32),
        pltpu.VMEM((NPT, 16), jnp.float32),
        pltpu.VMEM((NBUF, EC, F), jnp.float32),
        pltpu.VMEM((RC, F), jnp.float32),
        pltpu.VMEM_SHARED((N_PAD, F), jnp.float32),
        pltpu.VMEM_SHARED((N_PAD, F), jnp.float32),
        pltpu.SemaphoreType.DMA((NBUF,)),
        pltpu.SemaphoreType.DMA((NBUF,)),
        pltpu.SemaphoreType.DMA((2,)),
    ],
)
def _sc_prop(h_hbm, src_hbm, dst_hbm, g_out, deg_out, hd_hbm,
             srci, dsti, d2_t, rows_b, s_buf,
             g_sh, acc_sh, gsem, ssem, isem):
    cid = lax.axis_index("c")
    sid = lax.axis_index("s")
    base = sid * NPT
    own = (sid // 8) == cid

    zero16 = jnp.zeros((16,), jnp.float32)
    one16 = jnp.ones((16,), jnp.float32)

    def fill_buf(b, vec):
        def body(r, carry):
            for c in range(F // 16):
                rows_b[b, r, pl.ds(c * 16, 16)] = vec
            return carry
        lax.fori_loop(0, EC, body, 0)

    def fire_idx(slot, grp):
        j0 = grp * NBUF
        pltpu.async_copy(src_hbm.at[sid, pl.ds(j0, NBUF)], srci.at[slot],
                         isem.at[slot])
        pltpu.async_copy(dst_hbm.at[sid, pl.ds(j0, NBUF)], dsti.at[slot],
                         isem.at[slot])

    def wait_idx(slot):
        pltpu.make_async_copy(src_hbm.at[sid, pl.ds(0, NBUF)], srci.at[slot],
                              isem.at[slot]).wait()
        pltpu.make_async_copy(dst_hbm.at[sid, pl.ds(0, NBUF)], dsti.at[slot],
                              isem.at[slot]).wait()

    def process_group(slot, do_gather):
        if do_gather:
            for b in range(NBUF):
                pltpu.async_copy(g_sh.at[srci.at[slot, b]], rows_b.at[b],
                                 gsem.at[b])
        for b in range(NBUF):
            if do_gather:
                pltpu.make_async_copy(g_sh.at[srci.at[slot, b]],
                                      rows_b.at[b], gsem.at[b]).wait()
                sbuf = rows_b.at[b]
            else:
                sbuf = rows_b.at[1]
            pltpu.async_copy(sbuf, acc_sh.at[dsti.at[slot, b]], ssem.at[b],
                             add=True)
        for b in range(NBUF):
            pltpu.make_async_copy(rows_b.at[b], acc_sh.at[dsti.at[slot, 0]],
                                  ssem.at[b]).wait()

    def edge_sweep(do_gather):
        fire_idx(0, 0)

        def m_body(m, carry):
            fire_idx(1, 2 * m + 1)
            wait_idx(0)
            process_group(0, do_gather)

            @pl.when(m < EGRP // 2 - 1)
            def _():
                fire_idx(0, 2 * m + 2)
            wait_idx(1)
            process_group(1, do_gather)
            return carry
        lax.fori_loop(0, EGRP // 2, m_body, 0)

    fill_buf(0, zero16)
    for ch in range(NCH):
        pltpu.sync_copy(rows_b.at[0], acc_sh.at[pl.ds(base + ch * RC, RC)])
    plsc.subcore_barrier()

    fill_buf(1, one16)
    edge_sweep(False)
    plsc.subcore_barrier()

    for ch in range(NCH):
        cb = base + ch * RC
        pltpu.sync_copy(acc_sh.at[pl.ds(cb, RC)], s_buf)

        @pl.when(own)
        def _():
            pltpu.sync_copy(s_buf, deg_out.at[pl.ds(cb, RC)])
        pltpu.sync_copy(h_hbm.at[pl.ds(cb, RC)], rows_b.at[1])

        def prep_row(r, carry):
            degv = s_buf[r, pl.ds(0, 16)]
            y = _rsqrt16(degv)
            d2_t[ch * RC + r, pl.ds(0, 16)] = (1.0 - ALPHA) * y * y
            for c in range(F // 16):
                hv = rows_b[1, r, pl.ds(c * 16, 16)]
                g0 = y * hv
                rows_b[2, r, pl.ds(c * 16, 16)] = g0
                rows_b[1, r, pl.ds(c * 16, 16)] = ALPHA * g0
            return carry
        lax.fori_loop(0, RC, prep_row, 0)
        pltpu.sync_copy(rows_b.at[2], g_sh.at[pl.ds(cb, RC)])
        pltpu.sync_copy(rows_b.at[1], hd_hbm.at[pl.ds(cb, RC)])
        pltpu.sync_copy(rows_b.at[0], acc_sh.at[pl.ds(cb, RC)])
    plsc.subcore_barrier()

    def iter_body(k, carry):
        edge_sweep(True)
        plsc.subcore_barrier()

        not_last = k < K - 1

        @pl.when(not_last)
        def _():
            fill_buf(0, zero16)
        for ch in range(NCH):
            cb = base + ch * RC
            pltpu.sync_copy(acc_sh.at[pl.ds(cb, RC)], s_buf)
            pltpu.sync_copy(hd_hbm.at[pl.ds(cb, RC)], rows_b.at[1])

            def row_body(r, carry2):
                ddv = d2_t[ch * RC + r, pl.ds(0, 16)]
                for c in range(F // 16):
                    sv = s_buf[r, pl.ds(c * 16, 16)]
                    gv = ddv * sv + rows_b[1, r, pl.ds(c * 16, 16)]
                    s_buf[r, pl.ds(c * 16, 16)] = gv
                return carry2
            lax.fori_loop(0, RC, row_body, 0)

            @pl.when(not_last)
            def _():
                pltpu.sync_copy(s_buf, g_sh.at[pl.ds(cb, RC)])
                pltpu.sync_copy(rows_b.at[0], acc_sh.at[pl.ds(cb, RC)])

            @pl.when(jnp.logical_and(k == K - 1, own))
            def _():
                pltpu.sync_copy(s_buf, g_out.at[pl.ds(cb, RC)])
        plsc.subcore_barrier()
        return carry
    lax.fori_loop(0, K, iter_body, 0)


def _mlp(x, W1T, b1, W2T, b2):
    def body(x_ref, w1_ref, b1_ref, w2_ref, b2_ref, o_ref):
        a = jnp.dot(x_ref[...], w1_ref[...],
                    preferred_element_type=jnp.float32)
        a = jnp.maximum(a + b1_ref[...], 0.0)
        o_ref[...] = jnp.dot(a, w2_ref[...],
                             preferred_element_type=jnp.float32) + b2_ref[...]

    return pl.pallas_call(
        body,
        grid=(5,),
        in_specs=[
            pl.BlockSpec((2000, 128), lambda i: (i, 0)),
            pl.BlockSpec((128, 64), lambda i: (0, 0)),
            pl.BlockSpec((1, 64), lambda i: (0, 0)),
            pl.BlockSpec((64, 64), lambda i: (0, 0)),
            pl.BlockSpec((1, 64), lambda i: (0, 0)),
        ],
        out_specs=pl.BlockSpec((2000, 64), lambda i: (i, 0)),
        out_shape=jax.ShapeDtypeStruct((N, F), jnp.float32),
    )(x, W1T, b1.reshape(1, -1), W2T, b2.reshape(1, -1))


def _epilogue(g, deg):
    def body(g_ref, d_ref, o_ref):
        o_ref[...] = g_ref[...] * jnp.sqrt(d_ref[...])

    return pl.pallas_call(
        body,
        grid=(5,),
        in_specs=[
            pl.BlockSpec((N_PAD // 5, F), lambda i: (i, 0)),
            pl.BlockSpec((N_PAD // 5, F), lambda i: (i, 0)),
        ],
        out_specs=pl.BlockSpec((N_PAD // 5, F), lambda i: (i, 0)),
        out_shape=jax.ShapeDtypeStruct((N_PAD, F), jnp.float32),
    )(g, deg)


def kernel(x, edge_index, W1, b1, W2, b2):
    h = _mlp(x, W1.T, b1, W2.T, b2)
    h_pad = jnp.zeros((N_PAD, F), jnp.float32).at[:N].set(h)

    src = edge_index[0]
    dst = edge_index[1]
    loop = jnp.arange(N, dtype=jnp.int32)
    pad = E_SLOTS - src.shape[0] - N
    src_all = jnp.concatenate([src, loop, jnp.zeros((pad,), jnp.int32)])
    dst_all = jnp.concatenate([dst, loop, jnp.full((pad,), N, jnp.int32)])
    src_tiles = src_all.reshape(TILES, ECH, EC)
    dst_tiles = dst_all.reshape(TILES, ECH, EC)

    g_pad, deg_pad, _ = _sc_prop(h_pad, src_tiles, dst_tiles)
    out_pad = _epilogue(g_pad, deg_pad)
    return out_pad[:N]


# speedup vs baseline: 20.5156x; 1.1698x over previous
import functools

import jax
import jax.numpy as jnp
from jax import lax
from jax.experimental import pallas as pl
from jax.experimental.pallas import tpu as pltpu
from jax.experimental.pallas import tpu_sc as plsc

N = 10000
F = 64
K = 10
ALPHA = 0.1

N_PAD = 10240
NPT = 640
NCH = 5
RC = 128

TILES = 16
EC = 128
ECH = 162
NBUF = 3
EGRP = ECH // NBUF
E_SLOTS = TILES * ECH * EC

_mesh = plsc.VectorSubcoreMesh(core_axis_name="c", subcore_axis_name="s")


def _rsqrt16(v):
    ii = plsc.bitcast(v, jnp.int32)
    ii = 0x5F3759DF - lax.shift_right_logical(ii, 1)
    y = plsc.bitcast(ii, jnp.float32)
    for _ in range(3):
        y = y * (1.5 - 0.5 * v * y * y)
    return y


@functools.partial(
    pl.kernel,
    out_type=(
        jax.ShapeDtypeStruct((N_PAD, F), jnp.float32),
        jax.ShapeDtypeStruct((N_PAD, F), jnp.float32),
        jax.ShapeDtypeStruct((N_PAD, F), jnp.float32),
    ),
    mesh=_mesh,
    compiler_params=pltpu.CompilerParams(needs_layout_passes=False,
                                         use_tc_tiling_on_sc=False),
    scratch_types=[
        pltpu.VMEM((3, NBUF, EC), jnp.int32),
        pltpu.VMEM((3, NBUF, EC), jnp.int32),
        pltpu.VMEM((NPT, 16), jnp.float32),
        pltpu.VMEM((NBUF, EC, F), jnp.float32),
        pltpu.VMEM((RC, F), jnp.float32),
        pltpu.VMEM_SHARED((N_PAD, F), jnp.float32),
        pltpu.VMEM_SHARED((N_PAD, F), jnp.float32),
        pltpu.SemaphoreType.DMA((NBUF,)),
        pltpu.SemaphoreType.DMA((NBUF,)),
        pltpu.SemaphoreType.DMA((3,)),
    ],
)
def _sc_prop(h_hbm, src_hbm, dst_hbm, g_out, deg_out, hd_hbm,
             srci, dsti, d2_t, rows_b, s_buf,
             g_sh, acc_sh, gsem, ssem, isem):
    cid = lax.axis_index("c")
    sid = lax.axis_index("s")
    base = sid * NPT
    own = (sid // 8) == cid

    zero16 = jnp.zeros((16,), jnp.float32)
    one16 = jnp.ones((16,), jnp.float32)

    def fill_buf(b, vec):
        def body(r, carry):
            for c in range(F // 16):
                rows_b[b, r, pl.ds(c * 16, 16)] = vec
            return carry
        lax.fori_loop(0, EC, body, 0)

    def fire_idx(slot, grp):
        j0 = grp * NBUF
        pltpu.async_copy(src_hbm.at[sid, pl.ds(j0, NBUF)], srci.at[slot],
                         isem.at[slot])
        pltpu.async_copy(dst_hbm.at[sid, pl.ds(j0, NBUF)], dsti.at[slot],
                         isem.at[slot])

    def wait_idx(slot):
        pltpu.make_async_copy(src_hbm.at[sid, pl.ds(0, NBUF)], srci.at[slot],
                              isem.at[slot]).wait()
        pltpu.make_async_copy(dst_hbm.at[sid, pl.ds(0, NBUF)], dsti.at[slot],
                              isem.at[slot]).wait()

    def _wait_scatter(b):
        pltpu.make_async_copy(rows_b.at[b], acc_sh.at[dsti.at[0, 0]],
                              ssem.at[b]).wait()

    def _wait_gather(slot, pos, b):
        pltpu.make_async_copy(g_sh.at[srci.at[slot, pos]], rows_b.at[b],
                              gsem.at[b]).wait()

    def edge_sweep(do_gather):
        fire_idx(0, 0)
        fire_idx(1, 1)
        fire_idx(2, 2)

        def m_body(m, carry):
            for u in range(9):
                b = u % 3
                slot = u // 3
                pos = u % 3
                if u < 3:
                    @pl.when(m > 0)
                    def _():
                        _wait_scatter(b)
                else:
                    _wait_scatter(b)
                if pos == 0:
                    wait_idx(slot)
                if do_gather:
                    pltpu.async_copy(g_sh.at[srci.at[slot, pos]],
                                     rows_b.at[b], gsem.at[b])
                    bp = (u - 1) % 3
                    slotp = (u - 1) // 3 if u > 0 else 2
                    posp = (u - 1) % 3

                    def _prev_scatter():
                        _wait_gather(slotp, posp, bp)
                        pltpu.async_copy(rows_b.at[bp],
                                         acc_sh.at[dsti.at[slotp, posp]],
                                         ssem.at[bp], add=True)
                    if u == 0:
                        @pl.when(m > 0)
                        def _():
                            _prev_scatter()
                    else:
                        _prev_scatter()
                else:
                    pltpu.async_copy(rows_b.at[1],
                                     acc_sh.at[dsti.at[slot, pos]],
                                     ssem.at[b], add=True)
                if u == 2:
                    @pl.when(m > 0)
                    def _():
                        fire_idx(2, 3 * m + 2)
                elif u == 6:
                    @pl.when(m < EGRP // 3 - 1)
                    def _():
                        fire_idx(0, 3 * m + 3)
                elif u == 8:
                    @pl.when(m < EGRP // 3 - 1)
                    def _():
                        fire_idx(1, 3 * m + 4)
            return carry
        lax.fori_loop(0, EGRP // 3, m_body, 0)
        if do_gather:
            _wait_gather(2, 2, 2)
            pltpu.async_copy(rows_b.at[2], acc_sh.at[dsti.at[2, 2]],
                             ssem.at[2], add=True)
        for b in range(3):
            _wait_scatter(b)

    fill_buf(0, zero16)
    for ch in range(NCH):
        pltpu.sync_copy(rows_b.at[0], acc_sh.at[pl.ds(base + ch * RC, RC)])
    plsc.subcore_barrier()

    fill_buf(1, one16)
    edge_sweep(False)
    plsc.subcore_barrier()

    for ch in range(NCH):
        cb = base + ch * RC
        pltpu.sync_copy(acc_sh.at[pl.ds(cb, RC)], s_buf)

        @pl.when(own)
        def _():
            pltpu.sync_copy(s_buf, deg_out.at[pl.ds(cb, RC)])
        pltpu.sync_copy(h_hbm.at[pl.ds(cb, RC)], rows_b.at[1])

        def prep_row(r, carry):
            degv = s_buf[r, pl.ds(0, 16)]
            y = _rsqrt16(degv)
            d2_t[ch * RC + r, pl.ds(0, 16)] = (1.0 - ALPHA) * y * y
            for c in range(F // 16):
                hv = rows_b[1, r, pl.ds(c * 16, 16)]
                g0 = y * hv
                rows_b[2, r, pl.ds(c * 16, 16)] = g0
                rows_b[1, r, pl.ds(c * 16, 16)] = ALPHA * g0
            return carry
        lax.fori_loop(0, RC, prep_row, 0)
        pltpu.sync_copy(rows_b.at[2], g_sh.at[pl.ds(cb, RC)])
        pltpu.sync_copy(rows_b.at[1], hd_hbm.at[pl.ds(cb, RC)])
        pltpu.sync_copy(rows_b.at[0], acc_sh.at[pl.ds(cb, RC)])
    plsc.subcore_barrier()

    def iter_body(k, carry):
        edge_sweep(True)
        plsc.subcore_barrier()

        not_last = k < K - 1

        @pl.when(not_last)
        def _():
            fill_buf(0, zero16)
        for ch in range(NCH):
            cb = base + ch * RC
            pltpu.sync_copy(acc_sh.at[pl.ds(cb, RC)], s_buf)
            pltpu.sync_copy(hd_hbm.at[pl.ds(cb, RC)], rows_b.at[1])

            def row_body(r, carry2):
                ddv = d2_t[ch * RC + r, pl.ds(0, 16)]
                for c in range(F // 16):
                    sv = s_buf[r, pl.ds(c * 16, 16)]
                    gv = ddv * sv + rows_b[1, r, pl.ds(c * 16, 16)]
                    s_buf[r, pl.ds(c * 16, 16)] = gv
                return carry2
            lax.fori_loop(0, RC, row_body, 0)

            @pl.when(not_last)
            def _():
                pltpu.sync_copy(s_buf, g_sh.at[pl.ds(cb, RC)])
                pltpu.sync_copy(rows_b.at[0], acc_sh.at[pl.ds(cb, RC)])

            @pl.when(jnp.logical_and(k == K - 1, own))
            def _():
                pltpu.sync_copy(s_buf, g_out.at[pl.ds(cb, RC)])
        plsc.subcore_barrier()
        return carry
    lax.fori_loop(0, K, iter_body, 0)


def _mlp(x, W1T, b1, W2T, b2):
    def body(x_ref, w1_ref, b1_ref, w2_ref, b2_ref, o_ref):
        a = jnp.dot(x_ref[...], w1_ref[...],
                    preferred_element_type=jnp.float32)
        a = jnp.maximum(a + b1_ref[...], 0.0)
        o_ref[...] = jnp.dot(a, w2_ref[...],
                             preferred_element_type=jnp.float32) + b2_ref[...]

    return pl.pallas_call(
        body,
        grid=(5,),
        in_specs=[
            pl.BlockSpec((2000, 128), lambda i: (i, 0)),
            pl.BlockSpec((128, 64), lambda i: (0, 0)),
            pl.BlockSpec((1, 64), lambda i: (0, 0)),
            pl.BlockSpec((64, 64), lambda i: (0, 0)),
            pl.BlockSpec((1, 64), lambda i: (0, 0)),
        ],
        out_specs=pl.BlockSpec((2000, 64), lambda i: (i, 0)),
        out_shape=jax.ShapeDtypeStruct((N, F), jnp.float32),
    )(x, W1T, b1.reshape(1, -1), W2T, b2.reshape(1, -1))


def _epilogue(g, deg):
    def body(g_ref, d_ref, o_ref):
        o_ref[...] = g_ref[...] * jnp.sqrt(d_ref[...])

    return pl.pallas_call(
        body,
        grid=(5,),
        in_specs=[
            pl.BlockSpec((N_PAD // 5, F), lambda i: (i, 0)),
            pl.BlockSpec((N_PAD // 5, F), lambda i: (i, 0)),
        ],
        out_specs=pl.BlockSpec((N_PAD // 5, F), lambda i: (i, 0)),
        out_shape=jax.ShapeDtypeStruct((N_PAD, F), jnp.float32),
    )(g, deg)


def kernel(x, edge_index, W1, b1, W2, b2):
    h = _mlp(x, W1.T, b1, W2.T, b2)
    h_pad = jnp.zeros((N_PAD, F), jnp.float32).at[:N].set(h)

    src = edge_index[0]
    dst = edge_index[1]
    loop = jnp.arange(N, dtype=jnp.int32)
    pad = E_SLOTS - src.shape[0] - N
    src_all = jnp.concatenate([src, loop, jnp.zeros((pad,), jnp.int32)])
    dst_all = jnp.concatenate([dst, loop, jnp.full((pad,), N, jnp.int32)])
    src_tiles = src_all.reshape(TILES, ECH, EC)
    dst_tiles = dst_all.reshape(TILES, ECH, EC)

    g_pad, deg_pad, _ = _sc_prop(h_pad, src_tiles, dst_tiles)
    out_pad = _epilogue(g_pad, deg_pad)
    return out_pad[:N]


# speedup vs baseline: 29.5374x; 1.4398x over previous
import functools

import jax
import jax.numpy as jnp
from jax import lax
from jax.experimental import pallas as pl
from jax.experimental.pallas import tpu as pltpu
from jax.experimental.pallas import tpu_sc as plsc

N = 10000
F = 64
K = 10
ALPHA = 0.1

N_PAD = 10240
NPT = 640
NCH = 5
RC = 128

TILES = 16
EC = 128
ECH = 81
NBUF = 3
EGRP = ECH // NBUF
MB = ECH // 9
E_SLOTS = 2 * TILES * ECH * EC

_mesh = plsc.VectorSubcoreMesh(core_axis_name="c", subcore_axis_name="s")


def _rsqrt16(v):
    ii = plsc.bitcast(v, jnp.int32)
    ii = 0x5F3759DF - lax.shift_right_logical(ii, 1)
    y = plsc.bitcast(ii, jnp.float32)
    for _ in range(3):
        y = y * (1.5 - 0.5 * v * y * y)
    return y


@functools.partial(
    pl.kernel,
    out_type=(
        jax.ShapeDtypeStruct((N_PAD, F), jnp.float32),
        jax.ShapeDtypeStruct((N_PAD, F), jnp.float32),
        jax.ShapeDtypeStruct((N_PAD, F), jnp.float32),
        jax.ShapeDtypeStruct((2, N_PAD, F), jnp.float32),
    ),
    mesh=_mesh,
    compiler_params=pltpu.CompilerParams(needs_layout_passes=False,
                                         use_tc_tiling_on_sc=False),
    scratch_types=[
        pltpu.VMEM((3, NBUF, EC), jnp.int32),
        pltpu.VMEM((3, NBUF, EC), jnp.int32),
        pltpu.VMEM((NPT, 16), jnp.float32),
        pltpu.VMEM((NBUF, EC, F), jnp.float32),
        pltpu.VMEM((RC, F), jnp.float32),
        pltpu.VMEM_SHARED((N_PAD, F), jnp.float32),
        pltpu.VMEM_SHARED((N_PAD, F), jnp.float32),
        pltpu.SemaphoreType.DMA((NBUF,)),
        pltpu.SemaphoreType.DMA((NBUF,)),
        pltpu.SemaphoreType.DMA((3,)),
        pltpu.SemaphoreType.REGULAR,
    ],
)
def _sc_prop(h_hbm, src_hbm, dst_hbm, g_out, deg_out, hd_hbm, p_hbm,
             srci, dsti, d2_t, rows_b, s_buf,
             g_sh, acc_sh, gsem, ssem, isem, xsem):
    cid = lax.axis_index("c")
    sid = lax.axis_index("s")
    base = sid * NPT
    own = (sid // 8) == cid

    zero16 = jnp.zeros((16,), jnp.float32)
    one16 = jnp.ones((16,), jnp.float32)

    def fill_buf(b, vec):
        def body(r, carry):
            for c in range(F // 16):
                rows_b[b, r, pl.ds(c * 16, 16)] = vec
            return carry
        lax.fori_loop(0, EC, body, 0)

    def xbarrier():
        plsc.subcore_barrier()

        @pl.when(sid == 0)
        def _():
            pltpu.semaphore_signal(xsem, 1, core_index=1 - cid)
            pl.semaphore_wait(xsem, 1)
        plsc.subcore_barrier()

    def fire_idx(slot, grp):
        j0 = grp * NBUF
        pltpu.async_copy(src_hbm.at[cid, sid, pl.ds(j0, NBUF)],
                         srci.at[slot], isem.at[slot])
        pltpu.async_copy(dst_hbm.at[cid, sid, pl.ds(j0, NBUF)],
                         dsti.at[slot], isem.at[slot])

    def wait_idx(slot):
        pltpu.make_async_copy(src_hbm.at[cid, sid, pl.ds(0, NBUF)],
                              srci.at[slot], isem.at[slot]).wait()
        pltpu.make_async_copy(dst_hbm.at[cid, sid, pl.ds(0, NBUF)],
                              dsti.at[slot], isem.at[slot]).wait()

    def _wait_scatter(b):
        pltpu.make_async_copy(rows_b.at[b], acc_sh.at[dsti.at[0, 0]],
                              ssem.at[b]).wait()

    def _wait_gather(slot, pos, b):
        pltpu.make_async_copy(g_sh.at[srci.at[slot, pos]], rows_b.at[b],
                              gsem.at[b]).wait()

    def edge_sweep(do_gather):
        fire_idx(0, 0)
        fire_idx(1, 1)
        fire_idx(2, 2)

        def m_body(m, carry):
            for u in range(9):
                b = u % 3
                slot = u // 3
                pos = u % 3
                if u < 3:
                    @pl.when(m > 0)
                    def _():
                        _wait_scatter(b)
                else:
                    _wait_scatter(b)
                if pos == 0:
                    wait_idx(slot)
                if do_gather:
                    pltpu.async_copy(g_sh.at[srci.at[slot, pos]],
                                     rows_b.at[b], gsem.at[b])
                    bp = (u - 1) % 3
                    slotp = (u - 1) // 3 if u > 0 else 2
                    posp = (u - 1) % 3

                    def _prev_scatter():
                        _wait_gather(slotp, posp, bp)
                        pltpu.async_copy(rows_b.at[bp],
                                         acc_sh.at[dsti.at[slotp, posp]],
                                         ssem.at[bp], add=True)
                    if u == 0:
                        @pl.when(m > 0)
                        def _():
                            _prev_scatter()
                    else:
                        _prev_scatter()
                else:
                    pltpu.async_copy(rows_b.at[1],
                                     acc_sh.at[dsti.at[slot, pos]],
                                     ssem.at[b], add=True)
                if u == 2:
                    @pl.when(m > 0)
                    def _():
                        fire_idx(2, 3 * m + 2)
                elif u == 6:
                    @pl.when(m < MB - 1)
                    def _():
                        fire_idx(0, 3 * m + 3)
                elif u == 8:
                    @pl.when(m < MB - 1)
                    def _():
                        fire_idx(1, 3 * m + 4)
            return carry
        lax.fori_loop(0, MB, m_body, 0)
        if do_gather:
            _wait_gather(2, 2, 2)
            pltpu.async_copy(rows_b.at[2], acc_sh.at[dsti.at[2, 2]],
                             ssem.at[2], add=True)
        for b in range(3):
            _wait_scatter(b)

    def publish_partial():
        plsc.subcore_barrier()
        pltpu.sync_copy(acc_sh.at[pl.ds(base, NPT)],
                        p_hbm.at[cid, pl.ds(base, NPT)])
        xbarrier()

    fill_buf(0, zero16)
    for ch in range(NCH):
        pltpu.sync_copy(rows_b.at[0], acc_sh.at[pl.ds(base + ch * RC, RC)])
    plsc.subcore_barrier()

    fill_buf(1, one16)
    edge_sweep(False)
    publish_partial()

    fill_buf(0, zero16)
    for ch in range(NCH):
        cb = base + ch * RC
        pltpu.sync_copy(acc_sh.at[pl.ds(cb, RC)], s_buf)
        pltpu.sync_copy(p_hbm.at[1 - cid, pl.ds(cb, RC)], rows_b.at[2])

        def sum_row(r, carry):
            for c in range(F // 16):
                s_buf[r, pl.ds(c * 16, 16)] = (
                    s_buf[r, pl.ds(c * 16, 16)]
                    + rows_b[2, r, pl.ds(c * 16, 16)])
            return carry
        lax.fori_loop(0, RC, sum_row, 0)

        @pl.when(own)
        def _():
            pltpu.sync_copy(s_buf, deg_out.at[pl.ds(cb, RC)])
        pltpu.sync_copy(h_hbm.at[pl.ds(cb, RC)], rows_b.at[1])

        def prep_row(r, carry):
            degv = s_buf[r, pl.ds(0, 16)]
            y = _rsqrt16(degv)
            d2_t[ch * RC + r, pl.ds(0, 16)] = (1.0 - ALPHA) * y * y
            for c in range(F // 16):
                hv = rows_b[1, r, pl.ds(c * 16, 16)]
                g0 = y * hv
                rows_b[2, r, pl.ds(c * 16, 16)] = g0
                rows_b[1, r, pl.ds(c * 16, 16)] = ALPHA * g0
            return carry
        lax.fori_loop(0, RC, prep_row, 0)
        pltpu.sync_copy(rows_b.at[2], g_sh.at[pl.ds(cb, RC)])
        pltpu.sync_copy(rows_b.at[1], hd_hbm.at[pl.ds(cb, RC)])
        pltpu.sync_copy(rows_b.at[0], acc_sh.at[pl.ds(cb, RC)])
    xbarrier()

    def iter_body(k, carry):
        edge_sweep(True)
        publish_partial()

        not_last = k < K - 1

        @pl.when(not_last)
        def _():
            fill_buf(0, zero16)
        for ch in range(NCH):
            cb = base + ch * RC
            pltpu.sync_copy(acc_sh.at[pl.ds(cb, RC)], s_buf)
            pltpu.sync_copy(p_hbm.at[1 - cid, pl.ds(cb, RC)], rows_b.at[2])
            pltpu.sync_copy(hd_hbm.at[pl.ds(cb, RC)], rows_b.at[1])

            def row_body(r, carry2):
                ddv = d2_t[ch * RC + r, pl.ds(0, 16)]
                for c in range(F // 16):
                    sv = (s_buf[r, pl.ds(c * 16, 16)]
                          + rows_b[2, r, pl.ds(c * 16, 16)])
                    gv = ddv * sv + rows_b[1, r, pl.ds(c * 16, 16)]
                    s_buf[r, pl.ds(c * 16, 16)] = gv
                return carry2
            lax.fori_loop(0, RC, row_body, 0)

            @pl.when(not_last)
            def _():
                pltpu.sync_copy(s_buf, g_sh.at[pl.ds(cb, RC)])
                pltpu.sync_copy(rows_b.at[0], acc_sh.at[pl.ds(cb, RC)])

            @pl.when(jnp.logical_and(k == K - 1, own))
            def _():
                pltpu.sync_copy(s_buf, g_out.at[pl.ds(cb, RC)])
        xbarrier()
        return carry
    lax.fori_loop(0, K, iter_body, 0)


def _mlp(x, W1T, b1, W2T, b2):
    def body(x_ref, w1_ref, b1_ref, w2_ref, b2_ref, o_ref):
        a = jnp.dot(x_ref[...], w1_ref[...],
                    preferred_element_type=jnp.float32)
        a = jnp.maximum(a + b1_ref[...], 0.0)
        o_ref[...] = jnp.dot(a, w2_ref[...],
                             preferred_element_type=jnp.float32) + b2_ref[...]

    return pl.pallas_call(
        body,
        grid=(5,),
        in_specs=[
            pl.BlockSpec((2000, 128), lambda i: (i, 0)),
            pl.BlockSpec((128, 64), lambda i: (0, 0)),
            pl.BlockSpec((1, 64), lambda i: (0, 0)),
            pl.BlockSpec((64, 64), lambda i: (0, 0)),
            pl.BlockSpec((1, 64), lambda i: (0, 0)),
        ],
        out_specs=pl.BlockSpec((2000, 64), lambda i: (i, 0)),
        out_shape=jax.ShapeDtypeStruct((N, F), jnp.float32),
    )(x, W1T, b1.reshape(1, -1), W2T, b2.reshape(1, -1))


def _epilogue(g, deg):
    def body(g_ref, d_ref, o_ref):
        o_ref[...] = g_ref[...] * jnp.sqrt(d_ref[...])

    return pl.pallas_call(
        body,
        grid=(5,),
        in_specs=[
            pl.BlockSpec((N_PAD // 5, F), lambda i: (i, 0)),
            pl.BlockSpec((N_PAD // 5, F), lambda i: (i, 0)),
        ],
        out_specs=pl.BlockSpec((N_PAD // 5, F), lambda i: (i, 0)),
        out_shape=jax.ShapeDtypeStruct((N_PAD, F), jnp.float32),
    )(g, deg)


def kernel(x, edge_index, W1, b1, W2, b2):
    h = _mlp(x, W1.T, b1, W2.T, b2)
    h_pad = jnp.zeros((N_PAD, F), jnp.float32).at[:N].set(h)

    src = edge_index[0]
    dst = edge_index[1]
    loop = jnp.arange(N, dtype=jnp.int32)
    pad = E_SLOTS - src.shape[0] - N
    src_all = jnp.concatenate([src, loop, jnp.zeros((pad,), jnp.int32)])
    dst_all = jnp.concatenate([dst, loop, jnp.full((pad,), N, jnp.int32)])
    src_tiles = src_all.reshape(2, TILES, ECH, EC)
    dst_tiles = dst_all.reshape(2, TILES, ECH, EC)

    g_pad, deg_pad, _, _ = _sc_prop(h_pad, src_tiles, dst_tiles)
    out_pad = _epilogue(g_pad, deg_pad)
    return out_pad[:N]


# speedup vs baseline: 30.9300x; 1.0471x over previous
import functools

import jax
import jax.numpy as jnp
from jax import lax
from jax.experimental import pallas as pl
from jax.experimental.pallas import tpu as pltpu
from jax.experimental.pallas import tpu_sc as plsc

N = 10000
F = 64
K = 10
ALPHA = 0.1

N_PAD = 10240
NPT = 640
NCH = 5
RC = 128

TILES = 16
EC = 128
ECH = 81
NBUF = 3
EGRP = ECH // NBUF
MB = ECH // 9
E_SLOTS = 2 * TILES * ECH * EC

_mesh = plsc.VectorSubcoreMesh(core_axis_name="c", subcore_axis_name="s")


def _rsqrt16(v):
    ii = plsc.bitcast(v, jnp.int32)
    ii = 0x5F3759DF - lax.shift_right_logical(ii, 1)
    y = plsc.bitcast(ii, jnp.float32)
    for _ in range(3):
        y = y * (1.5 - 0.5 * v * y * y)
    return y


@functools.partial(
    pl.kernel,
    out_type=(
        jax.ShapeDtypeStruct((N_PAD, F), jnp.float32),
        jax.ShapeDtypeStruct((N_PAD, F), jnp.float32),
        jax.ShapeDtypeStruct((N_PAD, F), jnp.float32),
        jax.ShapeDtypeStruct((2, N_PAD, F), jnp.float32),
    ),
    mesh=_mesh,
    compiler_params=pltpu.CompilerParams(needs_layout_passes=False,
                                         use_tc_tiling_on_sc=False),
    scratch_types=[
        pltpu.VMEM((3, NBUF, EC), jnp.int32),
        pltpu.VMEM((3, NBUF, EC), jnp.int32),
        pltpu.VMEM((NPT, 16), jnp.float32),
        pltpu.VMEM((NBUF, EC, F), jnp.float32),
        pltpu.VMEM((RC, F), jnp.float32),
        pltpu.VMEM_SHARED((N_PAD, F), jnp.float32),
        pltpu.VMEM_SHARED((N_PAD, F), jnp.float32),
        pltpu.SemaphoreType.DMA((NBUF,)),
        pltpu.SemaphoreType.DMA((NBUF,)),
        pltpu.SemaphoreType.DMA((3,)),
        pltpu.SemaphoreType.REGULAR,
    ],
)
def _sc_prop(h_hbm, src_hbm, dst_hbm, g_out, deg_out, hd_hbm, p_hbm,
             srci, dsti, d2_t, rows_b, s_buf,
             g_sh, acc_sh, gsem, ssem, isem, xsem):
    cid = lax.axis_index("c")
    sid = lax.axis_index("s")
    base = sid * NPT
    own = (sid // 8) == cid

    zero16 = jnp.zeros((16,), jnp.float32)
    one16 = jnp.ones((16,), jnp.float32)

    def fill_buf(b, vec):
        def body(r, carry):
            for c in range(F // 16):
                rows_b[b, r, pl.ds(c * 16, 16)] = vec
            return carry
        lax.fori_loop(0, EC, body, 0)

    def xbarrier():
        plsc.subcore_barrier()

        @pl.when(sid == 0)
        def _():
            pltpu.semaphore_signal(xsem, 1, core_index=1 - cid)
            pl.semaphore_wait(xsem, 1)
        plsc.subcore_barrier()

    def fire_idx(slot, grp):
        j0 = grp * NBUF
        pltpu.async_copy(src_hbm.at[cid, sid, pl.ds(j0, NBUF)],
                         srci.at[slot], isem.at[slot])
        pltpu.async_copy(dst_hbm.at[cid, sid, pl.ds(j0, NBUF)],
                         dsti.at[slot], isem.at[slot])

    def wait_idx(slot):
        pltpu.make_async_copy(src_hbm.at[cid, sid, pl.ds(0, NBUF)],
                              srci.at[slot], isem.at[slot]).wait()
        pltpu.make_async_copy(dst_hbm.at[cid, sid, pl.ds(0, NBUF)],
                              dsti.at[slot], isem.at[slot]).wait()

    def _wait_scatter(b):
        pltpu.make_async_copy(rows_b.at[b], acc_sh.at[dsti.at[0, 0]],
                              ssem.at[b]).wait()

    def _wait_gather(slot, pos, b):
        pltpu.make_async_copy(g_sh.at[srci.at[slot, pos]], rows_b.at[b],
                              gsem.at[b]).wait()

    def edge_sweep(do_gather):
        fire_idx(0, 0)
        fire_idx(1, 1)
        fire_idx(2, 2)

        def m_body(m, carry):
            for u in range(9):
                b = u % 3
                slot = u // 3
                pos = u % 3
                if u < 3:
                    @pl.when(m > 0)
                    def _():
                        _wait_scatter(b)
                else:
                    _wait_scatter(b)
                if pos == 0:
                    wait_idx(slot)
                if do_gather:
                    pltpu.async_copy(g_sh.at[srci.at[slot, pos]],
                                     rows_b.at[b], gsem.at[b])
                    bp = (u - 1) % 3
                    slotp = (u - 1) // 3 if u > 0 else 2
                    posp = (u - 1) % 3

                    def _prev_scatter():
                        _wait_gather(slotp, posp, bp)
                        pltpu.async_copy(rows_b.at[bp],
                                         acc_sh.at[dsti.at[slotp, posp]],
                                         ssem.at[bp], add=True)
                    if u == 0:
                        @pl.when(m > 0)
                        def _():
                            _prev_scatter()
                    else:
                        _prev_scatter()
                else:
                    pltpu.async_copy(rows_b.at[1],
                                     acc_sh.at[dsti.at[slot, pos]],
                                     ssem.at[b], add=True)
                if u == 2:
                    @pl.when(m > 0)
                    def _():
                        fire_idx(2, 3 * m + 2)
                elif u == 6:
                    @pl.when(m < MB - 1)
                    def _():
                        fire_idx(0, 3 * m + 3)
                elif u == 8:
                    @pl.when(m < MB - 1)
                    def _():
                        fire_idx(1, 3 * m + 4)
            return carry
        lax.fori_loop(0, MB, m_body, 0)
        if do_gather:
            _wait_gather(2, 2, 2)
            pltpu.async_copy(rows_b.at[2], acc_sh.at[dsti.at[2, 2]],
                             ssem.at[2], add=True)
        for b in range(3):
            _wait_scatter(b)

    def publish_partial():
        plsc.subcore_barrier()
        pltpu.sync_copy(acc_sh.at[pl.ds(base, NPT)],
                        p_hbm.at[cid, pl.ds(base, NPT)])
        xbarrier()

    fill_buf(0, zero16)
    for ch in range(NCH):
        pltpu.sync_copy(rows_b.at[0], acc_sh.at[pl.ds(base + ch * RC, RC)])
    plsc.subcore_barrier()

    fill_buf(1, one16)
    edge_sweep(False)
    publish_partial()

    fill_buf(0, zero16)
    for ch in range(NCH):
        cb = base + ch * RC
        pltpu.sync_copy(acc_sh.at[pl.ds(cb, RC)], s_buf)
        pltpu.sync_copy(p_hbm.at[1 - cid, pl.ds(cb, RC)], rows_b.at[2])

        def sum_row(r, carry):
            for c in range(F // 16):
                s_buf[r, pl.ds(c * 16, 16)] = (
                    s_buf[r, pl.ds(c * 16, 16)]
                    + rows_b[2, r, pl.ds(c * 16, 16)])
            return carry
        lax.fori_loop(0, RC, sum_row, 0)

        @pl.when(own)
        def _():
            pltpu.sync_copy(s_buf, deg_out.at[pl.ds(cb, RC)])
        pltpu.sync_copy(h_hbm.at[pl.ds(cb, RC)], rows_b.at[1])

        def prep_row(r, carry):
            degv = s_buf[r, pl.ds(0, 16)]
            y = _rsqrt16(degv)
            d2_t[ch * RC + r, pl.ds(0, 16)] = (1.0 - ALPHA) * y * y
            for c in range(F // 16):
                hv = rows_b[1, r, pl.ds(c * 16, 16)]
                rows_b[2, r, pl.ds(c * 16, 16)] = y * hv
                rows_b[1, r, pl.ds(c * 16, 16)] = (
                    (ALPHA / (1.0 - ALPHA)) * (degv * y) * hv)
            return carry
        lax.fori_loop(0, RC, prep_row, 0)
        pltpu.sync_copy(rows_b.at[2], g_sh.at[pl.ds(cb, RC)])
        pltpu.sync_copy(rows_b.at[1], hd_hbm.at[pl.ds(cb, RC)])

        @pl.when(cid == 0)
        def _():
            pltpu.sync_copy(rows_b.at[1], acc_sh.at[pl.ds(cb, RC)])

        @pl.when(cid == 1)
        def _():
            pltpu.sync_copy(rows_b.at[0], acc_sh.at[pl.ds(cb, RC)])
    xbarrier()

    def iter_body(k, carry):
        edge_sweep(True)
        publish_partial()

        not_last = k < K - 1

        @pl.when(jnp.logical_and(not_last, cid == 1))
        def _():
            fill_buf(0, zero16)
        for ch in range(NCH):
            cb = base + ch * RC
            pltpu.sync_copy(acc_sh.at[pl.ds(cb, RC)], s_buf)
            pltpu.sync_copy(p_hbm.at[1 - cid, pl.ds(cb, RC)], rows_b.at[2])

            def row_body(r, carry2):
                ddv = d2_t[ch * RC + r, pl.ds(0, 16)]
                for c in range(F // 16):
                    sv = (s_buf[r, pl.ds(c * 16, 16)]
                          + rows_b[2, r, pl.ds(c * 16, 16)])
                    s_buf[r, pl.ds(c * 16, 16)] = ddv * sv
                return carry2
            lax.fori_loop(0, RC, row_body, 0)

            @pl.when(not_last)
            def _():
                pltpu.sync_copy(s_buf, g_sh.at[pl.ds(cb, RC)])

            @pl.when(jnp.logical_and(not_last, cid == 0))
            def _():
                pltpu.sync_copy(hd_hbm.at[pl.ds(cb, RC)],
                                acc_sh.at[pl.ds(cb, RC)])

            @pl.when(jnp.logical_and(not_last, cid == 1))
            def _():
                pltpu.sync_copy(rows_b.at[0], acc_sh.at[pl.ds(cb, RC)])

            @pl.when(jnp.logical_and(k == K - 1, own))
            def _():
                pltpu.sync_copy(s_buf, g_out.at[pl.ds(cb, RC)])
        xbarrier()
        return carry
    lax.fori_loop(0, K, iter_body, 0)


def _mlp(x, W1T, b1, W2T, b2):
    def body(x_ref, w1_ref, b1_ref, w2_ref, b2_ref, o_ref):
        a = jnp.dot(x_ref[...], w1_ref[...],
                    preferred_element_type=jnp.float32)
        a = jnp.maximum(a + b1_ref[...], 0.0)
        o_ref[...] = jnp.dot(a, w2_ref[...],
                             preferred_element_type=jnp.float32) + b2_ref[...]

    return pl.pallas_call(
        body,
        grid=(5,),
        in_specs=[
            pl.BlockSpec((2000, 128), lambda i: (i, 0)),
            pl.BlockSpec((128, 64), lambda i: (0, 0)),
            pl.BlockSpec((1, 64), lambda i: (0, 0)),
            pl.BlockSpec((64, 64), lambda i: (0, 0)),
            pl.BlockSpec((1, 64), lambda i: (0, 0)),
        ],
        out_specs=pl.BlockSpec((2000, 64), lambda i: (i, 0)),
        out_shape=jax.ShapeDtypeStruct((N, F), jnp.float32),
    )(x, W1T, b1.reshape(1, -1), W2T, b2.reshape(1, -1))


def _epilogue(g, deg):
    def body(g_ref, d_ref, o_ref):
        o_ref[...] = g_ref[...] * jnp.sqrt(d_ref[...])

    return pl.pallas_call(
        body,
        grid=(5,),
        in_specs=[
            pl.BlockSpec((N_PAD // 5, F), lambda i: (i, 0)),
            pl.BlockSpec((N_PAD // 5, F), lambda i: (i, 0)),
        ],
        out_specs=pl.BlockSpec((N_PAD // 5, F), lambda i: (i, 0)),
        out_shape=jax.ShapeDtypeStruct((N_PAD, F), jnp.float32),
    )(g, deg)


def kernel(x, edge_index, W1, b1, W2, b2):
    h = _mlp(x, W1.T, b1, W2.T, b2)
    h_pad = jnp.zeros((N_PAD, F), jnp.float32).at[:N].set(h)

    src = edge_index[0]
    dst = edge_index[1]
    loop = jnp.arange(N, dtype=jnp.int32)
    pad = E_SLOTS - src.shape[0] - N
    src_all = jnp.concatenate([src, loop, jnp.zeros((pad,), jnp.int32)])
    dst_all = jnp.concatenate([dst, loop, jnp.full((pad,), N, jnp.int32)])
    src_tiles = src_all.reshape(2, TILES, ECH, EC)
    dst_tiles = dst_all.reshape(2, TILES, ECH, EC)

    g_pad, deg_pad, _, _ = _sc_prop(h_pad, src_tiles, dst_tiles)
    out_pad = _epilogue(g_pad, deg_pad)
    return out_pad[:N]
